# baseline (speedup 1.0000x reference)
import jax
import jax.numpy as jnp
from jax import lax
from jax.experimental import pallas as pl
from jax.experimental.pallas import tpu as pltpu

N = 10000
FANOUT = 32
D = 128
BLK = 400
NSTEP = N // BLK
NBUF = 3


def _body(src_hbm, dst_hbm, w1_ref, w2_ref, b_ref, out_hbm,
          dbuf, sbuf, obuf, dsem, ssem, osem):
    src_cp = pltpu.make_async_copy(src_hbm, sbuf, ssem)
    src_cp.start()
    for i in range(NBUF):
        pltpu.make_async_copy(
            dst_hbm.at[pl.ds(i * BLK, BLK)], dbuf.at[i], dsem.at[i]
        ).start()
    src_cp.wait()

    def step(i, carry):
        j = lax.rem(i, NBUF)
        pltpu.make_async_copy(
            dst_hbm.at[pl.ds(i * BLK, BLK)], dbuf.at[j], dsem.at[j]
        ).wait()
        agg = dbuf[j].sum(axis=1)
        out = (
            jnp.dot(sbuf[pl.ds(i * BLK, BLK)], w1_ref[...],
                    preferred_element_type=jnp.float32)
            + jnp.dot(agg, w2_ref[...], preferred_element_type=jnp.float32)
            + b_ref[...]
        )
        obuf[pl.ds(i * BLK, BLK), :] = out

        @pl.when(i + NBUF < NSTEP)
        def _prefetch():
            pltpu.make_async_copy(
                dst_hbm.at[pl.ds((i + NBUF) * BLK, BLK)], dbuf.at[j],
                dsem.at[j]
            ).start()

        pltpu.make_async_copy(
            obuf.at[pl.ds(i * BLK, BLK)], out_hbm.at[pl.ds(i * BLK, BLK)],
            osem
        ).start()
        return carry

    lax.fori_loop(0, NSTEP, step, 0)

    def drain(i, carry):
        pltpu.make_async_copy(
            obuf.at[pl.ds(i * BLK, BLK)], out_hbm.at[pl.ds(i * BLK, BLK)],
            osem
        ).wait()
        return carry

    lax.fori_loop(0, NSTEP, drain, 0)


def kernel(src_feature, dst_feature, W, b):
    n = src_feature.shape[0]
    w1 = W[:D]
    w2 = W[D:] * (1.0 / FANOUT)
    b2 = b.reshape(1, D)
    return pl.pallas_call(
        _body,
        in_specs=[
            pl.BlockSpec(memory_space=pl.ANY),
            pl.BlockSpec(memory_space=pl.ANY),
            pl.BlockSpec((D, D), lambda: (0, 0)),
            pl.BlockSpec((D, D), lambda: (0, 0)),
            pl.BlockSpec((1, D), lambda: (0, 0)),
        ],
        out_specs=pl.BlockSpec(memory_space=pl.ANY),
        out_shape=jax.ShapeDtypeStruct((n, D), jnp.float32),
        scratch_shapes=[
            pltpu.VMEM((NBUF, BLK, FANOUT, D), jnp.float32),
            pltpu.VMEM((N, D), jnp.float32),
            pltpu.VMEM((N, D), jnp.float32),
            pltpu.SemaphoreType.DMA((NBUF,)),
            pltpu.SemaphoreType.DMA,
            pltpu.SemaphoreType.DMA,
        ],
    )(src_feature, dst_feature, w1, w2, b2)
